# initial kernel scaffold (unmeasured)
import jax
import jax.numpy as jnp
from jax import lax
from jax.experimental import pallas as pl
from jax.experimental.pallas import tpu as pltpu

N_DEV = 4
SQ = 256
D = 1024
HQ = 8
DH = 128
SCALE = 0.08838834764831843


def kernel(x, Wq, Wo, Wk, Wv):
    def body(x_ref, wq_ref, wo_ref, wk_ref, wv_ref, out_ref,
             xg_ref, p_ref, rs_ref,
             ag_send_sems, ag_recv_sems, rs_send_sems, rs_recv_sems):
        my = lax.axis_index("i")
        left = (my - 1) % N_DEV
        right = (my + 1) % N_DEV

        barrier_sem = pltpu.get_barrier_semaphore()
        for nbr in (left, right):
            pl.semaphore_signal(
                barrier_sem, inc=1,
                device_id=(nbr,), device_id_type=pl.DeviceIdType.MESH,
            )
        pl.semaphore_wait(barrier_sem, 2)

        xg_ref[pl.ds(my, 1)] = x_ref[...]
        for h in range(N_DEV - 1):
            idx = (my - h) % N_DEV
            rdma = pltpu.make_async_remote_copy(
                src_ref=xg_ref.at[idx],
                dst_ref=xg_ref.at[idx],
                send_sem=ag_send_sems.at[h],
                recv_sem=ag_recv_sems.at[h],
                device_id=(right,),
                device_id_type=pl.DeviceIdType.MESH,
            )
            rdma.start()
            rdma.wait()

        wq = wq_ref[...]
        wk = wk_ref[...]
        wv = wv_ref[...]
        wo = wo_ref[...]
        for b in range(N_DEV):
            xb = xg_ref[b]
            q = jnp.dot(xb, wq, preferred_element_type=jnp.float32)
            k = jnp.dot(xb, wk, preferred_element_type=jnp.float32)
            v = jnp.dot(xb, wv, preferred_element_type=jnp.float32)
            cols = []
            for hh in range(HQ):
                qh = q[:, hh * DH:(hh + 1) * DH]
                kh = k[:, hh * DH:(hh + 1) * DH]
                vh = v[:, hh * DH:(hh + 1) * DH]
                s = lax.dot_general(
                    qh, kh, (((1,), (1,)), ((), ())),
                    preferred_element_type=jnp.float32,
                ) * SCALE
                m = jnp.max(s, axis=1, keepdims=True)
                p = jnp.exp(s - m)
                l = jnp.sum(p, axis=1, keepdims=True)
                cols.append(jnp.dot(p, vh, preferred_element_type=jnp.float32) / l)
            attn = jnp.concatenate(cols, axis=1)
            p_ref[b] = jnp.dot(attn, wo, preferred_element_type=jnp.float32)

        for t in range(N_DEV - 1):
            c_send = (my - 1 - t) % N_DEV
            c_recv = (my - 2 - t) % N_DEV
            rdma = pltpu.make_async_remote_copy(
                src_ref=p_ref.at[c_send],
                dst_ref=rs_ref.at[t],
                send_sem=rs_send_sems.at[t],
                recv_sem=rs_recv_sems.at[t],
                device_id=(right,),
                device_id_type=pl.DeviceIdType.MESH,
            )
            rdma.start()
            rdma.wait()
            p_ref[pl.ds(c_recv, 1)] = p_ref[pl.ds(c_recv, 1)] + rs_ref[pl.ds(t, 1)]

        out_ref[...] = p_ref[pl.ds(my, 1)]

    return pl.pallas_call(
        body,
        out_shape=jax.ShapeDtypeStruct((1, SQ, D), jnp.float32),
        in_specs=[pl.BlockSpec(memory_space=pltpu.VMEM)] * 5,
        out_specs=pl.BlockSpec(memory_space=pltpu.VMEM),
        scratch_shapes=[
            pltpu.VMEM((N_DEV, SQ, D), jnp.float32),
            pltpu.VMEM((N_DEV, SQ, D), jnp.float32),
            pltpu.VMEM((N_DEV - 1, SQ, D), jnp.float32),
            pltpu.SemaphoreType.DMA((N_DEV - 1,)),
            pltpu.SemaphoreType.DMA((N_DEV - 1,)),
            pltpu.SemaphoreType.DMA((N_DEV - 1,)),
            pltpu.SemaphoreType.DMA((N_DEV - 1,)),
        ],
        compiler_params=pltpu.CompilerParams(collective_id=0),
    )(x, Wq, Wk, Wv, Wo)


# baseline (device time: 104165 ns/iter reference)
import jax
import jax.numpy as jnp
from jax import lax
from jax.experimental import pallas as pl
from jax.experimental.pallas import tpu as pltpu

N_DEV = 4
SQ = 256
D = 1024
HQ = 8
DH = 128
SCALE = 0.08838834764831843


def kernel(x, Wq, Wo, Wk, Wv):
    def body(x_ref, wq_ref, wk_ref, wv_ref, wo_ref, out_ref,
             xg_ref, p_ref, rs_ref,
             ag_send_sems, ag_recv_sems, rs_send_sems, rs_recv_sems):
        my = lax.axis_index("i")
        left = (my - 1) % N_DEV
        right = (my + 1) % N_DEV

        barrier_sem = pltpu.get_barrier_semaphore()
        for nbr in (left, right):
            pl.semaphore_signal(
                barrier_sem, inc=1,
                device_id=(nbr,), device_id_type=pl.DeviceIdType.MESH,
            )
        pl.semaphore_wait(barrier_sem, 2)

        xg_ref[pl.ds(my, 1)] = x_ref[...]
        for h in range(N_DEV - 1):
            idx = (my - h) % N_DEV
            rdma = pltpu.make_async_remote_copy(
                src_ref=xg_ref.at[idx],
                dst_ref=xg_ref.at[idx],
                send_sem=ag_send_sems.at[h],
                recv_sem=ag_recv_sems.at[h],
                device_id=(right,),
                device_id_type=pl.DeviceIdType.MESH,
            )
            rdma.start()
            rdma.wait()

        wq = wq_ref[...]
        wk = wk_ref[...]
        wv = wv_ref[...]
        wo = wo_ref[...]
        for b in range(N_DEV):
            xb = xg_ref[b]
            q = jnp.dot(xb, wq, preferred_element_type=jnp.float32)
            k = jnp.dot(xb, wk, preferred_element_type=jnp.float32)
            v = jnp.dot(xb, wv, preferred_element_type=jnp.float32)
            cols = []
            for hh in range(HQ):
                qh = q[:, hh * DH:(hh + 1) * DH]
                kh = k[:, hh * DH:(hh + 1) * DH]
                vh = v[:, hh * DH:(hh + 1) * DH]
                s = lax.dot_general(
                    qh, kh, (((1,), (1,)), ((), ())),
                    preferred_element_type=jnp.float32,
                ) * SCALE
                m = jnp.max(s, axis=1, keepdims=True)
                p = jnp.exp(s - m)
                l = jnp.sum(p, axis=1, keepdims=True)
                cols.append(jnp.dot(p, vh, preferred_element_type=jnp.float32) / l)
            attn = jnp.concatenate(cols, axis=1)
            p_ref[b] = jnp.dot(attn, wo, preferred_element_type=jnp.float32)

        for t in range(N_DEV - 1):
            c_send = (my - 1 - t) % N_DEV
            c_recv = (my - 2 - t) % N_DEV
            rdma = pltpu.make_async_remote_copy(
                src_ref=p_ref.at[c_send],
                dst_ref=rs_ref.at[t],
                send_sem=rs_send_sems.at[t],
                recv_sem=rs_recv_sems.at[t],
                device_id=(right,),
                device_id_type=pl.DeviceIdType.MESH,
            )
            rdma.start()
            rdma.wait()
            p_ref[pl.ds(c_recv, 1)] = p_ref[pl.ds(c_recv, 1)] + rs_ref[pl.ds(t, 1)]

        out_ref[...] = p_ref[pl.ds(my, 1)]

    return pl.pallas_call(
        body,
        out_shape=jax.ShapeDtypeStruct((1, SQ, D), jnp.float32),
        in_specs=[pl.BlockSpec(memory_space=pltpu.VMEM)] * 5,
        out_specs=pl.BlockSpec(memory_space=pltpu.VMEM),
        scratch_shapes=[
            pltpu.VMEM((N_DEV, SQ, D), jnp.float32),
            pltpu.VMEM((N_DEV, SQ, D), jnp.float32),
            pltpu.VMEM((N_DEV - 1, SQ, D), jnp.float32),
            pltpu.SemaphoreType.DMA((N_DEV - 1,)),
            pltpu.SemaphoreType.DMA((N_DEV - 1,)),
            pltpu.SemaphoreType.DMA((N_DEV - 1,)),
            pltpu.SemaphoreType.DMA((N_DEV - 1,)),
        ],
        compiler_params=pltpu.CompilerParams(collective_id=0),
    )(x, Wq, Wk, Wv, Wo)


# device time: 88176 ns/iter; 1.1813x vs baseline; 1.1813x over previous
import jax
import jax.numpy as jnp
from jax import lax
from jax.experimental import pallas as pl
from jax.experimental.pallas import tpu as pltpu

N_DEV = 4
SQ = 256
D = 1024
HQ = 8
DH = 128
SCALE = 0.08838834764831843


def kernel(x, Wq, Wo, Wk, Wv):
    def body(x_ref, wq_ref, wk_ref, wv_ref, wo_ref, out_ref,
             xg_ref, p_ref, rs_ref,
             ag_send_sems, ag_recv_sems, rs_send_sems, rs_recv_sems):
        my = lax.axis_index("i")
        left = (my - 1) % N_DEV
        right = (my + 1) % N_DEV

        barrier_sem = pltpu.get_barrier_semaphore()
        for nbr in (left, right):
            pl.semaphore_signal(
                barrier_sem, inc=1,
                device_id=(nbr,), device_id_type=pl.DeviceIdType.MESH,
            )
        pl.semaphore_wait(barrier_sem, 2)

        wq = wq_ref[...]
        wk = wk_ref[...]
        wv = wv_ref[...]
        wo = wo_ref[...]

        def compute_partial(xb):
            q = jnp.dot(xb, wq, preferred_element_type=jnp.float32)
            k = jnp.dot(xb, wk, preferred_element_type=jnp.float32)
            v = jnp.dot(xb, wv, preferred_element_type=jnp.float32)
            cols = []
            for hh in range(HQ):
                qh = q[:, hh * DH:(hh + 1) * DH]
                kh = k[:, hh * DH:(hh + 1) * DH]
                vh = v[:, hh * DH:(hh + 1) * DH]
                s = lax.dot_general(
                    qh, kh, (((1,), (1,)), ((), ())),
                    preferred_element_type=jnp.float32,
                ) * SCALE
                m = jnp.max(s, axis=1, keepdims=True)
                pj = jnp.exp(s - m)
                l = jnp.sum(pj, axis=1, keepdims=True)
                cols.append(jnp.dot(pj, vh, preferred_element_type=jnp.float32) / l)
            attn = jnp.concatenate(cols, axis=1)
            return jnp.dot(attn, wo, preferred_element_type=jnp.float32)

        def ag_rdma(h):
            idx = (my - h) % N_DEV
            return pltpu.make_async_remote_copy(
                src_ref=xg_ref.at[idx],
                dst_ref=xg_ref.at[idx],
                send_sem=ag_send_sems.at[h],
                recv_sem=ag_recv_sems.at[h],
                device_id=(right,),
                device_id_type=pl.DeviceIdType.MESH,
            )

        def rs_rdma(t):
            c_send = (my - 1 - t) % N_DEV
            return pltpu.make_async_remote_copy(
                src_ref=p_ref.at[c_send],
                dst_ref=rs_ref.at[t],
                send_sem=rs_send_sems.at[t],
                recv_sem=rs_recv_sems.at[t],
                device_id=(right,),
                device_id_type=pl.DeviceIdType.MESH,
            )

        xg_ref[pl.ds(my, 1)] = x_ref[...]
        ag0 = ag_rdma(0)
        ag0.start()

        p_ref[pl.ds(my, 1)] = compute_partial(x_ref[0])[None]

        ag0.wait_recv()
        ag1 = ag_rdma(1)
        ag1.start()
        c1 = (my - 1) % N_DEV
        p_ref[pl.ds(c1, 1)] = compute_partial(xg_ref[pl.ds(c1, 1)][0])[None]
        rs0 = rs_rdma(0)
        rs0.start()

        ag1.wait_recv()
        ag2 = ag_rdma(2)
        ag2.start()
        c2 = (my - 2) % N_DEV
        p_ref[pl.ds(c2, 1)] = compute_partial(xg_ref[pl.ds(c2, 1)][0])[None]
        rs0.wait_recv()
        p_ref[pl.ds(c2, 1)] = p_ref[pl.ds(c2, 1)] + rs_ref[pl.ds(0, 1)]
        rs1 = rs_rdma(1)
        rs1.start()

        ag2.wait_recv()
        c3 = (my - 3) % N_DEV
        p_ref[pl.ds(c3, 1)] = compute_partial(xg_ref[pl.ds(c3, 1)][0])[None]
        rs1.wait_recv()
        p_ref[pl.ds(c3, 1)] = p_ref[pl.ds(c3, 1)] + rs_ref[pl.ds(1, 1)]
        rs2 = rs_rdma(2)
        rs2.start()

        rs2.wait_recv()
        out_ref[...] = p_ref[pl.ds(my, 1)] + rs_ref[pl.ds(2, 1)]

        for r in (ag0, ag1, ag2, rs0, rs1, rs2):
            r.wait_send()

    return pl.pallas_call(
        body,
        out_shape=jax.ShapeDtypeStruct((1, SQ, D), jnp.float32),
        in_specs=[pl.BlockSpec(memory_space=pltpu.VMEM)] * 5,
        out_specs=pl.BlockSpec(memory_space=pltpu.VMEM),
        scratch_shapes=[
            pltpu.VMEM((N_DEV, SQ, D), jnp.float32),
            pltpu.VMEM((N_DEV, SQ, D), jnp.float32),
            pltpu.VMEM((N_DEV - 1, SQ, D), jnp.float32),
            pltpu.SemaphoreType.DMA((N_DEV - 1,)),
            pltpu.SemaphoreType.DMA((N_DEV - 1,)),
            pltpu.SemaphoreType.DMA((N_DEV - 1,)),
            pltpu.SemaphoreType.DMA((N_DEV - 1,)),
        ],
        compiler_params=pltpu.CompilerParams(collective_id=0),
    )(x, Wq, Wk, Wv, Wo)


# device time: 54332 ns/iter; 1.9172x vs baseline; 1.6229x over previous
import jax
import jax.numpy as jnp
from jax import lax
from jax.experimental import pallas as pl
from jax.experimental.pallas import tpu as pltpu

N_DEV = 4
SQ = 256
D = 1024
HQ = 8
DH = 128
SCALE = 0.08838834764831843


def kernel(x, Wq, Wo, Wk, Wv):
    def body(x_ref, wq_ref, wk_ref, wv_ref, wo_ref, out_ref,
             xg_ref, p_ref, rs_ref,
             ag_send_sems, ag_recv_sems, rs_send_sems, rs_recv_sems):
        my = lax.axis_index("i")
        left = (my - 1) % N_DEV
        right = (my + 1) % N_DEV

        barrier_sem = pltpu.get_barrier_semaphore()
        for nbr in (left, right):
            pl.semaphore_signal(
                barrier_sem, inc=1,
                device_id=(nbr,), device_id_type=pl.DeviceIdType.MESH,
            )
        pl.semaphore_wait(barrier_sem, 2)

        wq = wq_ref[...].astype(jnp.bfloat16)
        wk = wk_ref[...].astype(jnp.bfloat16)
        wv = wv_ref[...].astype(jnp.bfloat16)
        wo = wo_ref[...].astype(jnp.bfloat16)

        def compute_partial(xb):
            q = jnp.dot(xb, wq,
                        preferred_element_type=jnp.float32).astype(jnp.bfloat16)
            k = jnp.dot(xb, wk,
                        preferred_element_type=jnp.float32).astype(jnp.bfloat16)
            v = jnp.dot(xb, wv,
                        preferred_element_type=jnp.float32).astype(jnp.bfloat16)
            cols = []
            for hh in range(HQ):
                qh = q[:, hh * DH:(hh + 1) * DH]
                kh = k[:, hh * DH:(hh + 1) * DH]
                vh = v[:, hh * DH:(hh + 1) * DH]
                s = lax.dot_general(
                    qh, kh, (((1,), (1,)), ((), ())),
                    preferred_element_type=jnp.float32,
                ) * SCALE
                m = jnp.max(s, axis=1, keepdims=True)
                pj = jnp.exp(s - m)
                l = jnp.sum(pj, axis=1, keepdims=True)
                o = jnp.dot(pj.astype(jnp.bfloat16), vh,
                            preferred_element_type=jnp.float32) / l
                cols.append(o)
            attn = jnp.concatenate(cols, axis=1).astype(jnp.bfloat16)
            return jnp.dot(attn, wo, preferred_element_type=jnp.float32)

        def ag_rdma(h):
            idx = (my - h) % N_DEV
            return pltpu.make_async_remote_copy(
                src_ref=xg_ref.at[idx],
                dst_ref=xg_ref.at[idx],
                send_sem=ag_send_sems.at[h],
                recv_sem=ag_recv_sems.at[h],
                device_id=(right,),
                device_id_type=pl.DeviceIdType.MESH,
            )

        def rs_rdma(t):
            c_send = (my - 1 - t) % N_DEV
            return pltpu.make_async_remote_copy(
                src_ref=p_ref.at[c_send],
                dst_ref=rs_ref.at[t],
                send_sem=rs_send_sems.at[t],
                recv_sem=rs_recv_sems.at[t],
                device_id=(right,),
                device_id_type=pl.DeviceIdType.MESH,
            )

        f32 = jnp.float32
        bf16 = jnp.bfloat16

        xg_ref[pl.ds(my, 1)] = x_ref[...].astype(bf16)
        ag0 = ag_rdma(0)
        ag0.start()

        p_ref[pl.ds(my, 1)] = compute_partial(
            x_ref[0].astype(bf16))[None].astype(bf16)

        ag0.wait_recv()
        ag1 = ag_rdma(1)
        ag1.start()
        c1 = (my - 1) % N_DEV
        p_ref[pl.ds(c1, 1)] = compute_partial(
            xg_ref[pl.ds(c1, 1)][0])[None].astype(bf16)
        rs0 = rs_rdma(0)
        rs0.start()

        ag1.wait_recv()
        ag2 = ag_rdma(2)
        ag2.start()
        c2 = (my - 2) % N_DEV
        p2 = compute_partial(xg_ref[pl.ds(c2, 1)][0])
        rs0.wait_recv()
        p_ref[pl.ds(c2, 1)] = (
            p2[None] + rs_ref[pl.ds(0, 1)].astype(f32)).astype(bf16)
        rs1 = rs_rdma(1)
        rs1.start()

        ag2.wait_recv()
        c3 = (my - 3) % N_DEV
        p3 = compute_partial(xg_ref[pl.ds(c3, 1)][0])
        rs1.wait_recv()
        p_ref[pl.ds(c3, 1)] = (
            p3[None] + rs_ref[pl.ds(1, 1)].astype(f32)).astype(bf16)
        rs2 = rs_rdma(2)
        rs2.start()

        rs2.wait_recv()
        out_ref[...] = (p_ref[pl.ds(my, 1)].astype(f32)
                        + rs_ref[pl.ds(2, 1)].astype(f32))

        for r in (ag0, ag1, ag2, rs0, rs1, rs2):
            r.wait_send()

    return pl.pallas_call(
        body,
        out_shape=jax.ShapeDtypeStruct((1, SQ, D), jnp.float32),
        in_specs=[pl.BlockSpec(memory_space=pltpu.VMEM)] * 5,
        out_specs=pl.BlockSpec(memory_space=pltpu.VMEM),
        scratch_shapes=[
            pltpu.VMEM((N_DEV, SQ, D), jnp.bfloat16),
            pltpu.VMEM((N_DEV, SQ, D), jnp.bfloat16),
            pltpu.VMEM((N_DEV - 1, SQ, D), jnp.bfloat16),
            pltpu.SemaphoreType.DMA((N_DEV - 1,)),
            pltpu.SemaphoreType.DMA((N_DEV - 1,)),
            pltpu.SemaphoreType.DMA((N_DEV - 1,)),
            pltpu.SemaphoreType.DMA((N_DEV - 1,)),
        ],
        compiler_params=pltpu.CompilerParams(collective_id=0),
    )(x, Wq, Wk, Wv, Wo)


# device time: 45606 ns/iter; 2.2840x vs baseline; 1.1913x over previous
import jax
import jax.numpy as jnp
from jax import lax
from jax.experimental import pallas as pl
from jax.experimental.pallas import tpu as pltpu

N_DEV = 4
SQ = 256
D = 1024
HQ = 8
DH = 128
SCALE = 0.08838834764831843


def kernel(x, Wq, Wo, Wk, Wv):
    def body(x_ref, wq_ref, wk_ref, wv_ref, wo_ref, out_ref,
             xg_ref, p_ref, rs_ref,
             ag_send_sems, ag_recv_sems, rs_send_sems, rs_recv_sems):
        my = lax.axis_index("i")
        left = (my - 1) % N_DEV
        right = (my + 1) % N_DEV
        diag = (my + 2) % N_DEV
        peers = (left, right, diag)

        barrier_sem = pltpu.get_barrier_semaphore()
        for nbr in peers:
            pl.semaphore_signal(
                barrier_sem, inc=1,
                device_id=(nbr,), device_id_type=pl.DeviceIdType.MESH,
            )
        pl.semaphore_wait(barrier_sem, N_DEV - 1)

        wq = wq_ref[...].astype(jnp.bfloat16)
        wk = wk_ref[...].astype(jnp.bfloat16)
        wv = wv_ref[...].astype(jnp.bfloat16)
        wo = wo_ref[...].astype(jnp.bfloat16)

        def compute_partial(xb):
            q = jnp.dot(xb, wq,
                        preferred_element_type=jnp.float32).astype(jnp.bfloat16)
            k = jnp.dot(xb, wk,
                        preferred_element_type=jnp.float32).astype(jnp.bfloat16)
            v = jnp.dot(xb, wv,
                        preferred_element_type=jnp.float32).astype(jnp.bfloat16)
            cols = []
            for hh in range(HQ):
                qh = q[:, hh * DH:(hh + 1) * DH]
                kh = k[:, hh * DH:(hh + 1) * DH]
                vh = v[:, hh * DH:(hh + 1) * DH]
                s = lax.dot_general(
                    qh, kh, (((1,), (1,)), ((), ())),
                    preferred_element_type=jnp.float32,
                ) * SCALE
                m = jnp.max(s, axis=1, keepdims=True)
                pj = jnp.exp(s - m)
                l = jnp.sum(pj, axis=1, keepdims=True)
                o = jnp.dot(pj.astype(jnp.bfloat16), vh,
                            preferred_element_type=jnp.float32) / l
                cols.append(o)
            attn = jnp.concatenate(cols, axis=1).astype(jnp.bfloat16)
            return jnp.dot(attn, wo, preferred_element_type=jnp.float32)

        xg_ref[pl.ds(my, 1)] = x_ref[...].astype(jnp.bfloat16)
        ag_sends = []
        for k_, q_ in enumerate(peers):
            r = pltpu.make_async_remote_copy(
                src_ref=xg_ref.at[my],
                dst_ref=xg_ref.at[my],
                send_sem=ag_send_sems.at[k_],
                recv_sem=ag_recv_sems.at[my],
                device_id=(q_,),
                device_id_type=pl.DeviceIdType.MESH,
            )
            r.start()
            ag_sends.append(r)

        def ag_recv(q_):
            return pltpu.make_async_remote_copy(
                src_ref=xg_ref.at[q_], dst_ref=xg_ref.at[q_],
                send_sem=ag_send_sems.at[0], recv_sem=ag_recv_sems.at[q_],
                device_id=(q_,), device_id_type=pl.DeviceIdType.MESH,
            )

        rs_ref[pl.ds(my, 1)] = compute_partial(
            x_ref[0].astype(jnp.bfloat16))[None].astype(jnp.bfloat16)

        rs_sends = []
        for k_, q_ in enumerate((left, diag, right)):
            ag_recv(q_).wait_recv()
            p_ref[pl.ds(q_, 1)] = compute_partial(
                xg_ref[pl.ds(q_, 1)][0])[None].astype(jnp.bfloat16)
            r = pltpu.make_async_remote_copy(
                src_ref=p_ref.at[q_],
                dst_ref=rs_ref.at[my],
                send_sem=rs_send_sems.at[k_],
                recv_sem=rs_recv_sems.at[my],
                device_id=(q_,),
                device_id_type=pl.DeviceIdType.MESH,
            )
            r.start()
            rs_sends.append(r)

        for q_ in peers:
            pltpu.make_async_remote_copy(
                src_ref=p_ref.at[q_], dst_ref=rs_ref.at[q_],
                send_sem=rs_send_sems.at[0], recv_sem=rs_recv_sems.at[q_],
                device_id=(q_,), device_id_type=pl.DeviceIdType.MESH,
            ).wait_recv()
        out_ref[...] = jnp.sum(
            rs_ref[...].astype(jnp.float32), axis=0, keepdims=True)

        for r in ag_sends + rs_sends:
            r.wait_send()

    return pl.pallas_call(
        body,
        out_shape=jax.ShapeDtypeStruct((1, SQ, D), jnp.float32),
        in_specs=[pl.BlockSpec(memory_space=pltpu.VMEM)] * 5,
        out_specs=pl.BlockSpec(memory_space=pltpu.VMEM),
        scratch_shapes=[
            pltpu.VMEM((N_DEV, SQ, D), jnp.bfloat16),
            pltpu.VMEM((N_DEV, SQ, D), jnp.bfloat16),
            pltpu.VMEM((N_DEV, SQ, D), jnp.bfloat16),
            pltpu.SemaphoreType.DMA((N_DEV - 1,)),
            pltpu.SemaphoreType.DMA((N_DEV,)),
            pltpu.SemaphoreType.DMA((N_DEV - 1,)),
            pltpu.SemaphoreType.DMA((N_DEV,)),
        ],
        compiler_params=pltpu.CompilerParams(collective_id=0),
    )(x, Wq, Wk, Wv, Wo)


# device time: 45581 ns/iter; 2.2853x vs baseline; 1.0005x over previous
import jax
import jax.numpy as jnp
from jax import lax
from jax.experimental import pallas as pl
from jax.experimental.pallas import tpu as pltpu

N_DEV = 4
SQ = 256
D = 1024
HQ = 8
DH = 128
SCALE = 0.08838834764831843


def kernel(x, Wq, Wo, Wk, Wv):
    def body(x_ref, wq_ref, wk_ref, wv_ref, wo_ref, out_ref,
             xg_ref, p_ref, rs_ref,
             ag_send_sems, ag_recv_sems, rs_send_sems, rs_recv_sems):
        my = lax.axis_index("i")
        left = (my - 1) % N_DEV
        right = (my + 1) % N_DEV
        diag = (my + 2) % N_DEV
        peers = (left, right, diag)

        barrier_sem = pltpu.get_barrier_semaphore()
        for nbr in peers:
            pl.semaphore_signal(
                barrier_sem, inc=1,
                device_id=(nbr,), device_id_type=pl.DeviceIdType.MESH,
            )
        pl.semaphore_wait(barrier_sem, N_DEV - 1)

        bf16 = jnp.bfloat16
        f32 = jnp.float32
        wq = (wq_ref[...] * SCALE).astype(bf16)
        wk = wk_ref[...].astype(bf16)
        wv = wv_ref[...].astype(bf16)
        wo = wo_ref[...].astype(bf16)

        def compute_partial(xb, h0, nh):
            qs = jnp.dot(xb, wq[:, h0 * DH:(h0 + nh) * DH],
                         preferred_element_type=f32).astype(bf16)
            ks = jnp.dot(xb, wk[:, h0 * DH:(h0 + nh) * DH],
                         preferred_element_type=f32).astype(bf16)
            vs = jnp.dot(xb, wv[:, h0 * DH:(h0 + nh) * DH],
                         preferred_element_type=f32).astype(bf16)
            cols = []
            for hh in range(nh):
                qh = qs[:, hh * DH:(hh + 1) * DH]
                kh = ks[:, hh * DH:(hh + 1) * DH]
                vh = vs[:, hh * DH:(hh + 1) * DH]
                s = lax.dot_general(
                    qh, kh, (((1,), (1,)), ((), ())),
                    preferred_element_type=f32)
                pj = jnp.exp(s)
                l_inv = 1.0 / jnp.sum(pj, axis=1, keepdims=True)
                o = jnp.dot(pj.astype(bf16), vh,
                            preferred_element_type=f32) * l_inv
                cols.append(o)
            attn = jnp.concatenate(cols, axis=1).astype(bf16)
            return jnp.dot(attn, wo[h0 * DH:(h0 + nh) * DH, :],
                           preferred_element_type=f32)

        xg_ref[pl.ds(my, 1)] = x_ref[...].astype(bf16)
        ag_sends = []
        for k_, q_ in enumerate(peers):
            r = pltpu.make_async_remote_copy(
                src_ref=xg_ref.at[my],
                dst_ref=xg_ref.at[my],
                send_sem=ag_send_sems.at[k_],
                recv_sem=ag_recv_sems.at[my],
                device_id=(q_,),
                device_id_type=pl.DeviceIdType.MESH,
            )
            r.start()
            ag_sends.append(r)

        def ag_recv(q_):
            return pltpu.make_async_remote_copy(
                src_ref=xg_ref.at[q_], dst_ref=xg_ref.at[q_],
                send_sem=ag_send_sems.at[0], recv_sem=ag_recv_sems.at[q_],
                device_id=(q_,), device_id_type=pl.DeviceIdType.MESH,
            )

        xb_own = x_ref[0].astype(bf16)
        rs_ref[pl.ds(my, 1)] = compute_partial(xb_own, 0, HQ // 2)[None].astype(bf16)

        rs_sends = []
        for k_, q_ in enumerate((left, diag, right)):
            ag_recv(q_).wait_recv()
            p_ref[pl.ds(q_, 1)] = compute_partial(
                xg_ref[pl.ds(q_, 1)][0], 0, HQ)[None].astype(bf16)
            r = pltpu.make_async_remote_copy(
                src_ref=p_ref.at[q_],
                dst_ref=rs_ref.at[my],
                send_sem=rs_send_sems.at[k_],
                recv_sem=rs_recv_sems.at[my],
                device_id=(q_,),
                device_id_type=pl.DeviceIdType.MESH,
            )
            r.start()
            rs_sends.append(r)

        own_hi = compute_partial(xb_own, HQ // 2, HQ // 2)

        for q_ in peers:
            pltpu.make_async_remote_copy(
                src_ref=p_ref.at[q_], dst_ref=rs_ref.at[q_],
                send_sem=rs_send_sems.at[0], recv_sem=rs_recv_sems.at[q_],
                device_id=(q_,), device_id_type=pl.DeviceIdType.MESH,
            ).wait_recv()
        out_ref[...] = (
            jnp.sum(rs_ref[...].astype(f32), axis=0, keepdims=True)
            + own_hi[None])

        for r in ag_sends + rs_sends:
            r.wait_send()

    return pl.pallas_call(
        body,
        out_shape=jax.ShapeDtypeStruct((1, SQ, D), jnp.float32),
        in_specs=[pl.BlockSpec(memory_space=pltpu.VMEM)] * 5,
        out_specs=pl.BlockSpec(memory_space=pltpu.VMEM),
        scratch_shapes=[
            pltpu.VMEM((N_DEV, SQ, D), jnp.bfloat16),
            pltpu.VMEM((N_DEV, SQ, D), jnp.bfloat16),
            pltpu.VMEM((N_DEV, SQ, D), jnp.bfloat16),
            pltpu.SemaphoreType.DMA((N_DEV - 1,)),
            pltpu.SemaphoreType.DMA((N_DEV,)),
            pltpu.SemaphoreType.DMA((N_DEV - 1,)),
            pltpu.SemaphoreType.DMA((N_DEV,)),
        ],
        compiler_params=pltpu.CompilerParams(collective_id=0),
    )(x, Wq, Wk, Wv, Wo)


# device time: 34169 ns/iter; 3.0485x vs baseline; 1.3340x over previous
import jax
import jax.numpy as jnp
from jax import lax
from jax.experimental import pallas as pl
from jax.experimental.pallas import tpu as pltpu

N_DEV = 4
SQ = 256
D = 1024
HQ = 8
DH = 128
SCALE = 0.08838834764831843


def kernel(x, Wq, Wo, Wk, Wv):
    def body(x_ref, wq_ref, wk_ref, wv_ref, wo_ref, out_ref,
             o_vmem, xg_ref, p_ref, rs_ref, wv_vmem,
             ag_send_sems, ag_recv_sems, rs_send_sems, rs_recv_sems,
             w_sems):
        my = lax.axis_index("i")
        left = (my - 1) % N_DEV
        right = (my + 1) % N_DEV
        diag = (my + 2) % N_DEV
        peers = (left, right, diag)

        w_dmas = []
        for i_, wref in enumerate((wq_ref, wk_ref, wv_ref, wo_ref)):
            dma = pltpu.make_async_copy(wref, wv_vmem.at[i_], w_sems.at[i_])
            dma.start()
            w_dmas.append(dma)

        barrier_sem = pltpu.get_barrier_semaphore()
        for nbr in peers:
            pl.semaphore_signal(
                barrier_sem, inc=1,
                device_id=(nbr,), device_id_type=pl.DeviceIdType.MESH,
            )
        pl.semaphore_wait(barrier_sem, N_DEV - 1)

        bf16 = jnp.bfloat16
        f32 = jnp.float32
        w_cast = [None] * 4

        def get_w(i_):
            if w_cast[i_] is None:
                w_dmas[i_].wait()
                w = wv_vmem[i_]
                if i_ == 0:
                    w = w * SCALE
                w_cast[i_] = w.astype(bf16)
            return w_cast[i_]

        def compute_partial(xb, h0, nh):
            qs = jnp.dot(xb, get_w(0)[:, h0 * DH:(h0 + nh) * DH],
                         preferred_element_type=f32).astype(bf16)
            ks = jnp.dot(xb, get_w(1)[:, h0 * DH:(h0 + nh) * DH],
                         preferred_element_type=f32).astype(bf16)
            vs = jnp.dot(xb, get_w(2)[:, h0 * DH:(h0 + nh) * DH],
                         preferred_element_type=f32).astype(bf16)
            cols = []
            for hh in range(nh):
                qh = qs[:, hh * DH:(hh + 1) * DH]
                kh = ks[:, hh * DH:(hh + 1) * DH]
                vh = vs[:, hh * DH:(hh + 1) * DH]
                s = lax.dot_general(
                    qh, kh, (((1,), (1,)), ((), ())),
                    preferred_element_type=f32)
                pj = jnp.exp(s)
                l_inv = 1.0 / jnp.sum(pj, axis=1, keepdims=True)
                o = jnp.dot(pj.astype(bf16), vh,
                            preferred_element_type=f32) * l_inv
                cols.append(o)
            attn = jnp.concatenate(cols, axis=1).astype(bf16)
            return jnp.dot(attn, get_w(3)[h0 * DH:(h0 + nh) * DH, :],
                           preferred_element_type=f32)

        xg_ref[pl.ds(my, 1)] = x_ref[...].astype(bf16)
        ag_sends = []
        for k_, q_ in enumerate(peers):
            r = pltpu.make_async_remote_copy(
                src_ref=xg_ref.at[my],
                dst_ref=xg_ref.at[my],
                send_sem=ag_send_sems.at[k_],
                recv_sem=ag_recv_sems.at[my],
                device_id=(q_,),
                device_id_type=pl.DeviceIdType.MESH,
            )
            r.start()
            ag_sends.append(r)

        def ag_recv(q_):
            return pltpu.make_async_remote_copy(
                src_ref=xg_ref.at[q_], dst_ref=xg_ref.at[q_],
                send_sem=ag_send_sems.at[0], recv_sem=ag_recv_sems.at[q_],
                device_id=(q_,), device_id_type=pl.DeviceIdType.MESH,
            )

        xb_own = x_ref[0].astype(bf16)
        rs_ref[pl.ds(my, 1)] = compute_partial(xb_own, 0, HQ // 2)[None].astype(bf16)

        rs_sends = []
        for k_, q_ in enumerate((left, right, diag)):
            ag_recv(q_).wait_recv()
            p_ref[pl.ds(q_, 1)] = compute_partial(
                xg_ref[pl.ds(q_, 1)][0], 0, HQ)[None].astype(bf16)
            r = pltpu.make_async_remote_copy(
                src_ref=p_ref.at[q_],
                dst_ref=rs_ref.at[my],
                send_sem=rs_send_sems.at[k_],
                recv_sem=rs_recv_sems.at[my],
                device_id=(q_,),
                device_id_type=pl.DeviceIdType.MESH,
            )
            r.start()
            rs_sends.append(r)

        own_hi = compute_partial(xb_own, HQ // 2, HQ // 2)

        acc = own_hi + rs_ref[pl.ds(my, 1)][0].astype(f32)
        for q_ in (right, left, diag):
            pltpu.make_async_remote_copy(
                src_ref=p_ref.at[q_], dst_ref=rs_ref.at[q_],
                send_sem=rs_send_sems.at[0], recv_sem=rs_recv_sems.at[q_],
                device_id=(q_,), device_id_type=pl.DeviceIdType.MESH,
            ).wait_recv()
            acc = acc + rs_ref[pl.ds(q_, 1)][0].astype(f32)
        o_vmem[...] = acc[None]
        out_dma = pltpu.make_async_copy(o_vmem, out_ref, w_sems.at[0])
        out_dma.start()

        for r in ag_sends + rs_sends:
            r.wait_send()
        out_dma.wait()

    return pl.pallas_call(
        body,
        out_shape=jax.ShapeDtypeStruct((1, SQ, D), jnp.float32),
        in_specs=[pl.BlockSpec(memory_space=pltpu.VMEM)]
        + [pl.BlockSpec(memory_space=pl.ANY)] * 4,
        out_specs=pl.BlockSpec(memory_space=pl.ANY),
        scratch_shapes=[
            pltpu.VMEM((1, SQ, D), jnp.float32),
            pltpu.VMEM((N_DEV, SQ, D), jnp.bfloat16),
            pltpu.VMEM((N_DEV, SQ, D), jnp.bfloat16),
            pltpu.VMEM((N_DEV, SQ, D), jnp.bfloat16),
            pltpu.VMEM((4, D, D), jnp.float32),
            pltpu.SemaphoreType.DMA((N_DEV - 1,)),
            pltpu.SemaphoreType.DMA((N_DEV,)),
            pltpu.SemaphoreType.DMA((N_DEV - 1,)),
            pltpu.SemaphoreType.DMA((N_DEV,)),
            pltpu.SemaphoreType.DMA((4,)),
        ],
        compiler_params=pltpu.CompilerParams(
            collective_id=0, vmem_limit_bytes=100 * 1024 * 1024),
    )(x, Wq, Wk, Wv, Wo)
